# baseline (speedup 1.0000x reference)
import functools

import jax
import jax.numpy as jnp
from jax import lax
from jax.experimental import pallas as pl
from jax.experimental.pallas import tpu as pltpu
from jax.experimental.pallas import tpu_sc as plsc

DIM = 64
LANES = 16
NUM_CORES = 2
NUM_SUBCORES = 16
NW = NUM_CORES * NUM_SUBCORES

TOKENS = 4096 * 200
BPW = TOKENS // NW
CHUNK = 256
NG = BPW // CHUNK


def _sc_body(user_hbm, idx_hbm, table_hbm, out_hbm,
             idx_v, ebuf, ubuf, out_v, sem_e, sem_u):
    wid = lax.axis_index("s") * NUM_CORES + lax.axis_index("c")
    base = wid * BPW

    pltpu.sync_copy(idx_hbm.at[pl.ds(base, BPW)], idx_v)

    def chunk_body(g, _):
        cp_e = pltpu.async_copy(
            table_hbm.at[idx_v.at[pl.ds(g * CHUNK, CHUNK)]], ebuf, sem_e)
        cp_u = pltpu.async_copy(
            user_hbm.at[pl.ds(base + g * CHUNK, CHUNK)], ubuf, sem_u)
        cp_e.wait()
        cp_u.wait()

        lane = lax.iota(jnp.int32, LANES)

        def grp_body(q, _):
            tb = q * LANES
            acc = jnp.zeros((LANES,), jnp.float32)
            for t in range(LANES):
                s = ebuf[tb + t, pl.ds(0, LANES)] * ubuf[tb + t, pl.ds(0, LANES)]
                for k in range(1, DIM // LANES):
                    s += (ebuf[tb + t, pl.ds(k * LANES, LANES)]
                          * ubuf[tb + t, pl.ds(k * LANES, LANES)])
                acc = jnp.where(lane == t, jnp.sum(s), acc)
            out_v[pl.ds(g * CHUNK + tb, LANES)] = acc
            return 0

        lax.fori_loop(0, CHUNK // LANES, grp_body, 0)
        return 0

    lax.fori_loop(0, NG, chunk_body, 0)

    pltpu.sync_copy(out_v, out_hbm.at[pl.ds(base, BPW)])


@functools.partial(
    pl.kernel,
    mesh=plsc.VectorSubcoreMesh(core_axis_name="c", subcore_axis_name="s"),
    out_type=jax.ShapeDtypeStruct((TOKENS,), jnp.float32),
    compiler_params=pltpu.CompilerParams(
        needs_layout_passes=False, use_tc_tiling_on_sc=False),
    scratch_types=[
        pltpu.VMEM((BPW,), jnp.int32),
        pltpu.VMEM((CHUNK, DIM), jnp.float32),
        pltpu.VMEM((CHUNK, DIM), jnp.float32),
        pltpu.VMEM((BPW,), jnp.float32),
        pltpu.SemaphoreType.DMA,
        pltpu.SemaphoreType.DMA,
    ],
)
def _sc_kernel(user_hbm, idx_hbm, table_hbm, out_hbm,
               idx_v, ebuf, ubuf, out_v, sem_e, sem_u):
    _sc_body(user_hbm, idx_hbm, table_hbm, out_hbm,
             idx_v, ebuf, ubuf, out_v, sem_e, sem_u)


def kernel(user_rep, item_seq, item_emb_weight):
    u = user_rep.reshape(-1, DIM)
    idx = item_seq.reshape(-1)
    return _sc_kernel(u, idx, item_emb_weight)
